# baseline (speedup 1.0000x reference)
import functools

import jax
import jax.numpy as jnp
from jax import lax
from jax.experimental import pallas as pl
from jax.experimental.pallas import tpu as pltpu
from jax.experimental.pallas import tpu_sc as plsc

_B, _T, _D, _M = 16, 256, 256, 1024
_N = _B * _T
_R = 1024
_OFFS = (_D, 0, 2 * _D)


def _tc_body(emb_ref, en_ref, xa_ref, xna_ref, xv_ref, xnv_ref, xt_ref, xnt_ref,
             ia_ref, qa_ref, iv_ref, qv_ref, it_ref, qt_ref):
    emb = emb_ref[...]
    triples = ((xa_ref, xna_ref, ia_ref, qa_ref),
               (xv_ref, xnv_ref, iv_ref, qv_ref),
               (xt_ref, xnt_ref, it_ref, qt_ref))
    iota = lax.broadcasted_iota(jnp.int32, (_R, _M), 1)
    for m, (x_ref, xn_ref, i_ref, q_ref) in enumerate(triples):
        x = x_ref[...]
        e = emb[:, _OFFS[m]:_OFFS[m] + _D]
        xe = lax.dot_general(x, e, (((1,), (1,)), ((), ())),
                             preferred_element_type=jnp.float32)
        dist = (en_ref[m:m + 1, :] + xn_ref[...]) - 2.0 * xe
        mn = jnp.min(dist, axis=1, keepdims=True)
        idx = jnp.min(jnp.where(dist == mn, iota, _M), axis=1, keepdims=True)
        i_ref[...] = idx
        onehot = (iota == idx).astype(jnp.float32)
        g = lax.dot_general(onehot, e, (((1,), (0,)), ((), ())),
                            preferred_element_type=jnp.float32)
        q_ref[...] = x + (g - x)


_tc_call = pl.pallas_call(
    _tc_body,
    grid=(_N // _R,),
    in_specs=[
        pl.BlockSpec((_M, 3 * _D), lambda i: (0, 0)),
        pl.BlockSpec((3, _M), lambda i: (0, 0)),
        pl.BlockSpec((_R, _D), lambda i: (i, 0)),
        pl.BlockSpec((_R, 1), lambda i: (i, 0)),
        pl.BlockSpec((_R, _D), lambda i: (i, 0)),
        pl.BlockSpec((_R, 1), lambda i: (i, 0)),
        pl.BlockSpec((_R, _D), lambda i: (i, 0)),
        pl.BlockSpec((_R, 1), lambda i: (i, 0)),
    ],
    out_specs=[
        pl.BlockSpec((_R, 1), lambda i: (i, 0)),
        pl.BlockSpec((_R, _D), lambda i: (i, 0)),
        pl.BlockSpec((_R, 1), lambda i: (i, 0)),
        pl.BlockSpec((_R, _D), lambda i: (i, 0)),
        pl.BlockSpec((_R, 1), lambda i: (i, 0)),
        pl.BlockSpec((_R, _D), lambda i: (i, 0)),
    ],
    out_shape=[
        jax.ShapeDtypeStruct((_N, 1), jnp.int32),
        jax.ShapeDtypeStruct((_N, _D), jnp.float32),
        jax.ShapeDtypeStruct((_N, 1), jnp.int32),
        jax.ShapeDtypeStruct((_N, _D), jnp.float32),
        jax.ShapeDtypeStruct((_N, 1), jnp.int32),
        jax.ShapeDtypeStruct((_N, _D), jnp.float32),
    ],
)

_SC_ROWS = 128
_SC_CHUNK = 64


def _sc_gather_body(emb_hbm, ia_hbm, iv_hbm, it_hbm,
                    oa_hbm, ov_hbm, ot_hbm,
                    idx_v, rows0, rows1,
                    gsem0, gsem1, wsem0, wsem1):
    wid = lax.axis_index("s") * 2 + lax.axis_index("c")
    base = wid * _SC_ROWS
    ins = (ia_hbm, iv_hbm, it_hbm)
    outs = (oa_hbm, ov_hbm, ot_hbm)
    for m in range(3):
        pltpu.sync_copy(ins[m].at[pl.ds(base, _SC_ROWS)], idx_v.at[m])
    bufs = (rows0, rows1)
    gsems = (gsem0, gsem1)
    wsems = (wsem0, wsem1)
    chunks = [(m, c) for m in range(3) for c in range(2)]
    writes = [None, None]
    for k, (m, c) in enumerate(chunks):
        b = k % 2
        if writes[b] is not None:
            writes[b].wait()
        pltpu.async_copy(
            emb_hbm.at[idx_v.at[m, pl.ds(c * _SC_CHUNK, _SC_CHUNK)]],
            bufs[b], gsems[b]).wait()
        writes[b] = pltpu.async_copy(
            bufs[b], outs[m].at[pl.ds(base + c * _SC_CHUNK, _SC_CHUNK)],
            wsems[b])
    writes[0].wait()
    writes[1].wait()


def _make_sc_gather():
    return functools.partial(
        pl.kernel,
        out_type=(
            jax.ShapeDtypeStruct((_N, 3 * _D), jnp.float32),
            jax.ShapeDtypeStruct((_N, 3 * _D), jnp.float32),
            jax.ShapeDtypeStruct((_N, 3 * _D), jnp.float32),
        ),
        mesh=plsc.VectorSubcoreMesh(core_axis_name="c", subcore_axis_name="s"),
        scratch_types=[
            pltpu.VMEM((3, _SC_ROWS), jnp.int32),
            pltpu.VMEM((_SC_CHUNK, 3 * _D), jnp.float32),
            pltpu.VMEM((_SC_CHUNK, 3 * _D), jnp.float32),
            pltpu.SemaphoreType.DMA,
            pltpu.SemaphoreType.DMA,
            pltpu.SemaphoreType.DMA,
            pltpu.SemaphoreType.DMA,
        ],
    )(_sc_gather_body)


def kernel(audio_semantic, video_semantic, text_semantic, epoch, embedding):
    d = audio_semantic.shape[-1]
    flats = [jnp.reshape(x, (-1, d)) for x in (audio_semantic, video_semantic, text_semantic)]
    xns = [jnp.sum(f ** 2, axis=1, keepdims=True) for f in flats]
    slices = [embedding[:, _OFFS[m]:_OFFS[m] + d] for m in range(3)]
    en = jnp.stack([jnp.sum(e ** 2, axis=1) for e in slices])

    ia, qa, iv, qv, it, qt = _tc_call(
        embedding, en,
        flats[0], xns[0], flats[1], xns[1], flats[2], xns[2])

    oa, ov, ot = _make_sc_gather()(
        embedding,
        jnp.reshape(ia, (-1,)), jnp.reshape(iv, (-1,)), jnp.reshape(it, (-1,)))

    full = (_B, _T, 3 * d)
    part = (_B, _T, d)
    return (jnp.reshape(oa, full), jnp.reshape(qa, part),
            jnp.reshape(ov, full), jnp.reshape(qv, part),
            jnp.reshape(ot, full), jnp.reshape(qt, part))


# speedup vs baseline: 1.0057x; 1.0057x over previous
import functools

import jax
import jax.numpy as jnp
from jax import lax
from jax.experimental import pallas as pl
from jax.experimental.pallas import tpu as pltpu
from jax.experimental.pallas import tpu_sc as plsc

_B, _T, _D, _M = 16, 256, 256, 1024
_N = _B * _T
_R = 2048
_OFFS = (_D, 0, 2 * _D)


def _tc_body(emb_ref, en_ref, xa_ref, xna_ref, xv_ref, xnv_ref, xt_ref, xnt_ref,
             ia_ref, qa_ref, iv_ref, qv_ref, it_ref, qt_ref):
    emb = emb_ref[...]
    triples = ((xa_ref, xna_ref, ia_ref, qa_ref),
               (xv_ref, xnv_ref, iv_ref, qv_ref),
               (xt_ref, xnt_ref, it_ref, qt_ref))
    iota = lax.broadcasted_iota(jnp.int32, (_R, _M), 1)
    for m, (x_ref, xn_ref, i_ref, q_ref) in enumerate(triples):
        x = x_ref[...]
        e = emb[:, _OFFS[m]:_OFFS[m] + _D]
        xe = lax.dot_general(x, e, (((1,), (1,)), ((), ())),
                             preferred_element_type=jnp.float32)
        dist = (en_ref[m:m + 1, :] + xn_ref[...]) - 2.0 * xe
        mn = jnp.min(dist, axis=1, keepdims=True)
        idx = jnp.min(jnp.where(dist == mn, iota, _M), axis=1, keepdims=True)
        i_ref[...] = idx
        onehot = (iota == idx).astype(jnp.float32)
        g = lax.dot_general(onehot, e, (((1,), (0,)), ((), ())),
                            preferred_element_type=jnp.float32)
        q_ref[...] = x + (g - x)


_tc_call = pl.pallas_call(
    _tc_body,
    grid=(_N // _R,),
    in_specs=[
        pl.BlockSpec((_M, 3 * _D), lambda i: (0, 0)),
        pl.BlockSpec((3, _M), lambda i: (0, 0)),
        pl.BlockSpec((_R, _D), lambda i: (i, 0)),
        pl.BlockSpec((_R, 1), lambda i: (i, 0)),
        pl.BlockSpec((_R, _D), lambda i: (i, 0)),
        pl.BlockSpec((_R, 1), lambda i: (i, 0)),
        pl.BlockSpec((_R, _D), lambda i: (i, 0)),
        pl.BlockSpec((_R, 1), lambda i: (i, 0)),
    ],
    out_specs=[
        pl.BlockSpec((_R, 1), lambda i: (i, 0)),
        pl.BlockSpec((_R, _D), lambda i: (i, 0)),
        pl.BlockSpec((_R, 1), lambda i: (i, 0)),
        pl.BlockSpec((_R, _D), lambda i: (i, 0)),
        pl.BlockSpec((_R, 1), lambda i: (i, 0)),
        pl.BlockSpec((_R, _D), lambda i: (i, 0)),
    ],
    out_shape=[
        jax.ShapeDtypeStruct((_N, 1), jnp.int32),
        jax.ShapeDtypeStruct((_N, _D), jnp.float32),
        jax.ShapeDtypeStruct((_N, 1), jnp.int32),
        jax.ShapeDtypeStruct((_N, _D), jnp.float32),
        jax.ShapeDtypeStruct((_N, 1), jnp.int32),
        jax.ShapeDtypeStruct((_N, _D), jnp.float32),
    ],
)

_SC_CHUNK = 128


def _sc_gather_body(emb_hbm, ia_hbm, iv_hbm, it_hbm,
                    oa_hbm, ov_hbm, ot_hbm, idx_v, rows_v, sem):
    wid = lax.axis_index("s") * 2 + lax.axis_index("c")
    base = wid * _SC_CHUNK
    for idx_hbm, out_hbm in ((ia_hbm, oa_hbm), (iv_hbm, ov_hbm), (it_hbm, ot_hbm)):
        pltpu.sync_copy(idx_hbm.at[pl.ds(base, _SC_CHUNK)], idx_v)
        pltpu.async_copy(emb_hbm.at[idx_v], rows_v, sem).wait()
        pltpu.sync_copy(rows_v, out_hbm.at[pl.ds(base, _SC_CHUNK)])


def _make_sc_gather():
    return functools.partial(
        pl.kernel,
        out_type=(
            jax.ShapeDtypeStruct((_N, 3 * _D), jnp.float32),
            jax.ShapeDtypeStruct((_N, 3 * _D), jnp.float32),
            jax.ShapeDtypeStruct((_N, 3 * _D), jnp.float32),
        ),
        mesh=plsc.VectorSubcoreMesh(core_axis_name="c", subcore_axis_name="s"),
        scratch_types=[
            pltpu.VMEM((_SC_CHUNK,), jnp.int32),
            pltpu.VMEM((_SC_CHUNK, 3 * _D), jnp.float32),
            pltpu.SemaphoreType.DMA,
        ],
    )(_sc_gather_body)


def kernel(audio_semantic, video_semantic, text_semantic, epoch, embedding):
    d = audio_semantic.shape[-1]
    flats = [jnp.reshape(x, (-1, d)) for x in (audio_semantic, video_semantic, text_semantic)]
    xns = [jnp.sum(f ** 2, axis=1, keepdims=True) for f in flats]
    slices = [embedding[:, _OFFS[m]:_OFFS[m] + d] for m in range(3)]
    en = jnp.stack([jnp.sum(e ** 2, axis=1) for e in slices])

    ia, qa, iv, qv, it, qt = _tc_call(
        embedding, en,
        flats[0], xns[0], flats[1], xns[1], flats[2], xns[2])

    oa, ov, ot = _make_sc_gather()(
        embedding,
        jnp.reshape(ia, (-1,)), jnp.reshape(iv, (-1,)), jnp.reshape(it, (-1,)))

    full = (_B, _T, 3 * d)
    part = (_B, _T, d)
    return (jnp.reshape(oa, full), jnp.reshape(qa, part),
            jnp.reshape(ov, full), jnp.reshape(qv, part),
            jnp.reshape(ot, full), jnp.reshape(qt, part))


# speedup vs baseline: 1.1236x; 1.1172x over previous
import functools

import jax
import jax.numpy as jnp
from jax import lax
from jax.experimental import pallas as pl
from jax.experimental.pallas import tpu as pltpu
from jax.experimental.pallas import tpu_sc as plsc

_B, _T, _D, _M = 16, 256, 256, 1024
_N = _B * _T
_R = 1024
_OFFS = (_D, 0, 2 * _D)


def _tc_body(emb_ref, en_ref, xa_ref, xna_ref, xv_ref, xnv_ref, xt_ref, xnt_ref,
             ia_ref, qa_ref, iv_ref, qv_ref, it_ref, qt_ref):
    emb = emb_ref[...]
    triples = ((xa_ref, xna_ref, ia_ref, qa_ref),
               (xv_ref, xnv_ref, iv_ref, qv_ref),
               (xt_ref, xnt_ref, it_ref, qt_ref))
    iota = lax.broadcasted_iota(jnp.int32, (_M, _R), 0)
    for m, (x_ref, xn_ref, i_ref, q_ref) in enumerate(triples):
        x = x_ref[...]
        e = emb[:, _OFFS[m]:_OFFS[m] + _D]
        xe = lax.dot_general(e, x, (((1,), (1,)), ((), ())),
                             preferred_element_type=jnp.float32)
        dist = (en_ref[:, m:m + 1] + xn_ref[...]) - 2.0 * xe
        mn = jnp.min(dist, axis=0, keepdims=True)
        idx = jnp.min(jnp.where(dist == mn, iota, _M), axis=0, keepdims=True)
        i_ref[...] = idx
        onehot = (iota == idx).astype(jnp.float32)
        g = lax.dot_general(onehot, e, (((0,), (0,)), ((), ())),
                            preferred_element_type=jnp.float32)
        q_ref[...] = x + (g - x)


_tc_call = pl.pallas_call(
    _tc_body,
    grid=(_N // _R,),
    in_specs=[
        pl.BlockSpec((_M, 3 * _D), lambda i: (0, 0)),
        pl.BlockSpec((_M, 3), lambda i: (0, 0)),
        pl.BlockSpec((_R, _D), lambda i: (i, 0)),
        pl.BlockSpec((1, _R), lambda i: (0, i)),
        pl.BlockSpec((_R, _D), lambda i: (i, 0)),
        pl.BlockSpec((1, _R), lambda i: (0, i)),
        pl.BlockSpec((_R, _D), lambda i: (i, 0)),
        pl.BlockSpec((1, _R), lambda i: (0, i)),
    ],
    out_specs=[
        pl.BlockSpec((1, _R), lambda i: (0, i)),
        pl.BlockSpec((_R, _D), lambda i: (i, 0)),
        pl.BlockSpec((1, _R), lambda i: (0, i)),
        pl.BlockSpec((_R, _D), lambda i: (i, 0)),
        pl.BlockSpec((1, _R), lambda i: (0, i)),
        pl.BlockSpec((_R, _D), lambda i: (i, 0)),
    ],
    out_shape=[
        jax.ShapeDtypeStruct((1, _N), jnp.int32),
        jax.ShapeDtypeStruct((_N, _D), jnp.float32),
        jax.ShapeDtypeStruct((1, _N), jnp.int32),
        jax.ShapeDtypeStruct((_N, _D), jnp.float32),
        jax.ShapeDtypeStruct((1, _N), jnp.int32),
        jax.ShapeDtypeStruct((_N, _D), jnp.float32),
    ],
)

_SC_CHUNK = 128


def _sc_gather_body(emb_hbm, ia_hbm, iv_hbm, it_hbm,
                    oa_hbm, ov_hbm, ot_hbm, idx_v, rows_v, sem):
    wid = lax.axis_index("s") * 2 + lax.axis_index("c")
    base = wid * _SC_CHUNK
    for idx_hbm, out_hbm in ((ia_hbm, oa_hbm), (iv_hbm, ov_hbm), (it_hbm, ot_hbm)):
        pltpu.sync_copy(idx_hbm.at[pl.ds(base, _SC_CHUNK)], idx_v)
        pltpu.async_copy(emb_hbm.at[idx_v], rows_v, sem).wait()
        pltpu.sync_copy(rows_v, out_hbm.at[pl.ds(base, _SC_CHUNK)])


def _make_sc_gather():
    return functools.partial(
        pl.kernel,
        out_type=(
            jax.ShapeDtypeStruct((_N, 3 * _D), jnp.float32),
            jax.ShapeDtypeStruct((_N, 3 * _D), jnp.float32),
            jax.ShapeDtypeStruct((_N, 3 * _D), jnp.float32),
        ),
        mesh=plsc.VectorSubcoreMesh(core_axis_name="c", subcore_axis_name="s"),
        scratch_types=[
            pltpu.VMEM((_SC_CHUNK,), jnp.int32),
            pltpu.VMEM((_SC_CHUNK, 3 * _D), jnp.float32),
            pltpu.SemaphoreType.DMA,
        ],
    )(_sc_gather_body)


def kernel(audio_semantic, video_semantic, text_semantic, epoch, embedding):
    d = audio_semantic.shape[-1]
    flats = [jnp.reshape(x, (-1, d)) for x in (audio_semantic, video_semantic, text_semantic)]
    xns = [jnp.sum(f ** 2, axis=1)[None, :] for f in flats]
    slices = [embedding[:, _OFFS[m]:_OFFS[m] + d] for m in range(3)]
    en = jnp.stack([jnp.sum(e ** 2, axis=1) for e in slices], axis=1)

    ia, qa, iv, qv, it, qt = _tc_call(
        embedding, en,
        flats[0], xns[0], flats[1], xns[1], flats[2], xns[2])

    oa, ov, ot = _make_sc_gather()(
        embedding,
        jnp.reshape(ia, (-1,)), jnp.reshape(iv, (-1,)), jnp.reshape(it, (-1,)))

    full = (_B, _T, 3 * d)
    part = (_B, _T, d)
    return (jnp.reshape(oa, full), jnp.reshape(qa, part),
            jnp.reshape(ov, full), jnp.reshape(qv, part),
            jnp.reshape(ot, full), jnp.reshape(qt, part))
